# Initial kernel scaffold; baseline (speedup 1.0000x reference)
import jax
import jax.numpy as jnp
from jax.experimental import pallas as pl


def kernel(x, edge_index, W1, b1, W2, b2, Wfc, bfc):
    raise NotImplementedError("write your pallas kernel here")


# speedup vs baseline: 10.0592x; 10.0592x over previous
import functools

import jax
import jax.numpy as jnp
from jax import lax
from jax.experimental import pallas as pl
from jax.experimental.pallas import tpu as pltpu
from jax.experimental.pallas import tpu_sc as plsc

N_NODES = 10000
D_IN = 128
D_HID = 256
D_HALF = 128

NUM_CORES = 2
NUM_TILES = 16
CHUNK = 128

ACC_ROWS = 10240
SLAB = ACC_ROWS // NUM_TILES


def _sc_mesh():
    return plsc.VectorSubcoreMesh(
        core_axis_name="c", subcore_axis_name="s",
        num_cores=NUM_CORES, num_subcores=NUM_TILES)


def _sc_degree(col_p, e_pad):
    per_worker = e_pad // (NUM_CORES * NUM_TILES)
    n_chunks = per_worker // CHUNK

    @functools.partial(
        pl.kernel,
        out_type=jax.ShapeDtypeStruct((NUM_CORES * ACC_ROWS, D_HALF),
                                      jnp.float32),
        mesh=_sc_mesh(),
        scratch_types=[
            pltpu.VMEM((CHUNK,), jnp.int32),
            pltpu.VMEM((CHUNK, D_HALF), jnp.float32),
            pltpu.VMEM_SHARED((ACC_ROWS, D_HALF), jnp.float32),
        ],
    )
    def deg_kernel(col_hbm, out_hbm, cidx_v, ones_v, acc_sh):
        c = lax.axis_index("c")
        t = lax.axis_index("s")
        wid = c * NUM_TILES + t

        @pl.loop(0, CHUNK)
        def _(i):
            @pl.loop(0, D_HALF, step=16)
            def _(j):
                ones_v[i, pl.ds(j, 16)] = jnp.zeros((16,), jnp.float32)

        @pl.loop(0, SLAB // CHUNK)
        def _(j):
            pltpu.sync_copy(ones_v,
                            acc_sh.at[pl.ds(t * SLAB + j * CHUNK, CHUNK)])

        @pl.loop(0, CHUNK)
        def _(i):
            @pl.loop(0, D_HALF, step=16)
            def _(j):
                ones_v[i, pl.ds(j, 16)] = jnp.ones((16,), jnp.float32)
        plsc.subcore_barrier()

        @pl.loop(0, n_chunks)
        def _(k):
            base = wid * per_worker + k * CHUNK
            pltpu.sync_copy(col_hbm.at[pl.ds(base, CHUNK)], cidx_v)
            pltpu.sync_copy(ones_v, acc_sh.at[cidx_v], add=True)

        plsc.subcore_barrier()
        pltpu.sync_copy(
            acc_sh.at[pl.ds(t * SLAB, SLAB)],
            out_hbm.at[pl.ds(c * ACC_ROWS + t * SLAB, SLAB)])

    return deg_kernel(col_p)


def _sc_aggregate(y0, y1, row_p, col_p, e_pad):
    per_tile = e_pad // NUM_TILES
    n_chunks = per_tile // CHUNK

    @functools.partial(
        pl.kernel,
        out_type=(jax.ShapeDtypeStruct((ACC_ROWS, D_HALF), jnp.float32),
                  jax.ShapeDtypeStruct((ACC_ROWS, D_HALF), jnp.float32)),
        mesh=_sc_mesh(),
        scratch_types=[
            pltpu.VMEM((CHUNK,), jnp.int32),
            pltpu.VMEM((CHUNK,), jnp.int32),
            pltpu.VMEM((CHUNK, D_HALF), jnp.float32),
            pltpu.VMEM_SHARED((ACC_ROWS, D_HALF), jnp.float32),
            pltpu.SemaphoreType.DMA,
        ],
    )
    def agg_kernel(y0_hbm, y1_hbm, row_hbm, col_hbm, a0_hbm, a1_hbm,
                   ridx_v, cidx_v, rows_v, acc_sh, sem):
        c = lax.axis_index("c")
        t = lax.axis_index("s")

        @pl.loop(0, CHUNK)
        def _(i):
            @pl.loop(0, D_HALF, step=16)
            def _(j):
                rows_v[i, pl.ds(j, 16)] = jnp.zeros((16,), jnp.float32)

        @pl.loop(0, SLAB // CHUNK)
        def _(j):
            pltpu.sync_copy(rows_v,
                            acc_sh.at[pl.ds(t * SLAB + j * CHUNK, CHUNK)])
        plsc.subcore_barrier()

        def run(y_hbm):
            @pl.loop(0, n_chunks)
            def _(k):
                base = t * per_tile + k * CHUNK
                pltpu.sync_copy(row_hbm.at[pl.ds(base, CHUNK)], ridx_v)
                pltpu.sync_copy(col_hbm.at[pl.ds(base, CHUNK)], cidx_v)
                pltpu.async_copy(y_hbm.at[ridx_v], rows_v, sem).wait()
                pltpu.sync_copy(rows_v, acc_sh.at[cidx_v], add=True)

        @pl.when(c == 0)
        def _():
            run(y0_hbm)

        @pl.when(c == 1)
        def _():
            run(y1_hbm)

        plsc.subcore_barrier()

        def writeback(a_hbm):
            pltpu.sync_copy(acc_sh.at[pl.ds(t * SLAB, SLAB)],
                            a_hbm.at[pl.ds(t * SLAB, SLAB)])

        @pl.when(c == 0)
        def _():
            writeback(a0_hbm)

        @pl.when(c == 1)
        def _():
            writeback(a1_hbm)

    return agg_kernel(y0, y1, row_p, col_p)


_BLK = 1000
_GRID = N_NODES // _BLK

_DOT = functools.partial(jnp.dot, preferred_element_type=jnp.float32,
                         precision=lax.Precision.HIGHEST)


def _row_spec(width):
    return pl.BlockSpec((_BLK, width), lambda i: (i, 0))


def _full_spec(shape):
    return pl.BlockSpec(shape, lambda i: (0, 0))


def _tc_matmul(x, w):
    k_dim, m_dim = w.shape

    def body(x_ref, w_ref, o_ref):
        o_ref[...] = _DOT(x_ref[...], w_ref[...])

    return pl.pallas_call(
        body,
        grid=(_GRID,),
        in_specs=[_row_spec(k_dim), _full_spec((k_dim, m_dim))],
        out_specs=_row_spec(m_dim),
        out_shape=jax.ShapeDtypeStruct((N_NODES, m_dim), jnp.float32),
    )(x, w)


def _tc_scale_split(z, p0, p1):
    def body(z_ref, p0_ref, p1_ref, y0_ref, y1_ref):
        s = lax.rsqrt(p0_ref[:, :1] + p1_ref[:, :1] + 1.0)
        y = s * z_ref[...]
        y0_ref[...] = y[:, :D_HALF]
        y1_ref[...] = y[:, D_HALF:]

    return pl.pallas_call(
        body,
        grid=(_GRID,),
        in_specs=[_row_spec(D_HID), _row_spec(D_HALF), _row_spec(D_HALF)],
        out_specs=(_row_spec(D_HALF), _row_spec(D_HALF)),
        out_shape=(jax.ShapeDtypeStruct((N_NODES, D_HALF), jnp.float32),
                   jax.ShapeDtypeStruct((N_NODES, D_HALF), jnp.float32)),
    )(z, p0, p1)


def _tc_mid(a0, a1, y0, y1, p0, p1, b1, w2):
    def body(a0_ref, a1_ref, y0_ref, y1_ref, p0_ref, p1_ref, b_ref, w_ref,
             o0_ref, o1_ref):
        s = lax.rsqrt(p0_ref[:, :1] + p1_ref[:, :1] + 1.0)
        agg = jnp.concatenate([a0_ref[...], a1_ref[...]], axis=1)
        y = jnp.concatenate([y0_ref[...], y1_ref[...]], axis=1)
        h = jnp.maximum(s * (agg + y) + b_ref[...], 0.0)
        y2 = s * _DOT(h, w_ref[...])
        o0_ref[...] = y2[:, :D_HALF]
        o1_ref[...] = y2[:, D_HALF:]

    return pl.pallas_call(
        body,
        grid=(_GRID,),
        in_specs=[_row_spec(D_HALF), _row_spec(D_HALF),
                  _row_spec(D_HALF), _row_spec(D_HALF),
                  _row_spec(D_HALF), _row_spec(D_HALF),
                  _full_spec((1, D_HID)), _full_spec((D_HID, D_HID))],
        out_specs=(_row_spec(D_HALF), _row_spec(D_HALF)),
        out_shape=(jax.ShapeDtypeStruct((N_NODES, D_HALF), jnp.float32),
                   jax.ShapeDtypeStruct((N_NODES, D_HALF), jnp.float32)),
    )(a0, a1, y0, y1, p0, p1, b1, w2)


def _tc_final(a0, a1, y0, y1, p0, p1, b2, wfc_pad, bfc_pad):
    def body(a0_ref, a1_ref, y0_ref, y1_ref, p0_ref, p1_ref, b_ref, w_ref,
             bf_ref, o_ref):
        s = lax.rsqrt(p0_ref[:, :1] + p1_ref[:, :1] + 1.0)
        agg = jnp.concatenate([a0_ref[...], a1_ref[...]], axis=1)
        y = jnp.concatenate([y0_ref[...], y1_ref[...]], axis=1)
        h = jnp.maximum(s * (agg + y) + b_ref[...], 0.0)
        o_ref[...] = jax.nn.sigmoid(_DOT(h, w_ref[...]) + bf_ref[...])

    return pl.pallas_call(
        body,
        grid=(_GRID,),
        in_specs=[_row_spec(D_HALF), _row_spec(D_HALF),
                  _row_spec(D_HALF), _row_spec(D_HALF),
                  _row_spec(D_HALF), _row_spec(D_HALF),
                  _full_spec((1, D_HID)), _full_spec((D_HID, 128)),
                  _full_spec((1, 128))],
        out_specs=_row_spec(128),
        out_shape=jax.ShapeDtypeStruct((N_NODES, 128), jnp.float32),
    )(a0, a1, y0, y1, p0, p1, b2, wfc_pad, bfc_pad)


def kernel(x, edge_index, W1, b1, W2, b2, Wfc, bfc):
    n_edges = edge_index.shape[1]
    unit = NUM_CORES * NUM_TILES * CHUNK
    e_pad = -(-n_edges // unit) * unit
    pad_n = e_pad - n_edges

    row = edge_index[0].astype(jnp.int32)
    col = edge_index[1].astype(jnp.int32)
    pad_ids = jnp.arange(pad_n, dtype=jnp.int32) % 16
    row_p = jnp.concatenate([row, pad_ids])
    col_p = jnp.concatenate([col, N_NODES + pad_ids])

    p = _sc_degree(col_p, e_pad)
    p0 = p[:ACC_ROWS]
    p1 = p[ACC_ROWS:]

    z1 = _tc_matmul(x, W1)
    y1_0, y1_1 = _tc_scale_split(z1, p0, p1)
    a1_0, a1_1 = _sc_aggregate(y1_0, y1_1, row_p, col_p, e_pad)

    y2_0, y2_1 = _tc_mid(a1_0, a1_1, y1_0, y1_1, p0, p1,
                         b1.reshape(1, D_HID), W2)
    a2_0, a2_1 = _sc_aggregate(y2_0, y2_1, row_p, col_p, e_pad)

    wfc_pad = jnp.pad(Wfc, ((0, 0), (0, 128 - Wfc.shape[1])))
    bfc_pad = jnp.pad(bfc.reshape(1, 1), ((0, 0), (0, 127)))
    out = _tc_final(a2_0, a2_1, y2_0, y2_1, p0, p1,
                    b2.reshape(1, D_HID), wfc_pad, bfc_pad)
    return out[:, :1]


# speedup vs baseline: 20.6892x; 2.0567x over previous
import functools

import jax
import jax.numpy as jnp
from jax import lax
from jax.experimental import pallas as pl
from jax.experimental.pallas import tpu as pltpu
from jax.experimental.pallas import tpu_sc as plsc

N_NODES = 10000
D_IN = 128
D_HID = 256
D_HALF = 128

NUM_CORES = 2
NUM_TILES = 16
CHUNK = 128

ACC_ROWS = 10240
SLAB = ACC_ROWS // NUM_TILES


def _sc_mesh():
    return plsc.VectorSubcoreMesh(
        core_axis_name="c", subcore_axis_name="s",
        num_cores=NUM_CORES, num_subcores=NUM_TILES)


D_DEG = 16


def _sc_degree(col_p, e_pad):
    per_worker = e_pad // (NUM_CORES * NUM_TILES)
    n_chunks = per_worker // CHUNK

    @functools.partial(
        pl.kernel,
        out_type=jax.ShapeDtypeStruct((NUM_CORES * ACC_ROWS, D_DEG),
                                      jnp.float32),
        mesh=_sc_mesh(),
        scratch_types=[
            pltpu.VMEM((per_worker,), jnp.int32),
            pltpu.VMEM((CHUNK,), jnp.int32),
            pltpu.VMEM((CHUNK, D_DEG), jnp.float32),
            pltpu.VMEM_SHARED((ACC_ROWS, D_DEG), jnp.float32),
        ],
    )
    def deg_kernel(col_hbm, out_hbm, cidx_v, cchunk_v, ones_v, acc_sh):
        c = lax.axis_index("c")
        t = lax.axis_index("s")
        wid = c * NUM_TILES + t

        @pl.loop(0, CHUNK)
        def _(i):
            ones_v[i, pl.ds(0, D_DEG)] = jnp.zeros((D_DEG,), jnp.float32)

        @pl.loop(0, SLAB // CHUNK)
        def _(j):
            pltpu.sync_copy(ones_v,
                            acc_sh.at[pl.ds(t * SLAB + j * CHUNK, CHUNK)])

        @pl.loop(0, CHUNK)
        def _(i):
            ones_v[i, pl.ds(0, D_DEG)] = jnp.ones((D_DEG,), jnp.float32)

        pltpu.sync_copy(col_hbm.at[pl.ds(wid * per_worker, per_worker)],
                        cidx_v)
        plsc.subcore_barrier()

        @pl.loop(0, n_chunks)
        def _(k):
            @pl.loop(0, CHUNK, step=16)
            def _(i):
                cchunk_v[pl.ds(i, 16)] = cidx_v[pl.ds(k * CHUNK + i, 16)]

            pltpu.sync_copy(ones_v, acc_sh.at[cchunk_v], add=True)

        plsc.subcore_barrier()
        pltpu.sync_copy(
            acc_sh.at[pl.ds(t * SLAB, SLAB)],
            out_hbm.at[pl.ds(c * ACC_ROWS + t * SLAB, SLAB)])

    return deg_kernel(col_p)


NB = 16


def _sc_aggregate(y0, y1, rc3, e_pad):
    per_tile = e_pad // NUM_TILES
    n_chunks = per_tile // CHUNK
    n_blk = n_chunks // NB

    @functools.partial(
        pl.kernel,
        out_type=(jax.ShapeDtypeStruct((ACC_ROWS, D_HALF), jnp.float32),
                  jax.ShapeDtypeStruct((ACC_ROWS, D_HALF), jnp.float32)),
        mesh=_sc_mesh(),
        scratch_types=[
            pltpu.VMEM((NB, 2, CHUNK), jnp.int32),
            pltpu.VMEM((CHUNK,), jnp.int32),
            pltpu.VMEM((CHUNK,), jnp.int32),
            pltpu.VMEM((CHUNK,), jnp.int32),
            pltpu.VMEM((CHUNK,), jnp.int32),
            pltpu.VMEM((CHUNK, D_HALF), jnp.float32),
            pltpu.VMEM((CHUNK, D_HALF), jnp.float32),
            pltpu.VMEM_SHARED((ACC_ROWS, D_HALF), jnp.float32),
            pltpu.SemaphoreType.DMA,
            pltpu.SemaphoreType.DMA,
        ],
    )
    def agg_kernel(y0_hbm, y1_hbm, rc_hbm, a0_hbm, a1_hbm,
                   ib, r0_v, c0_v, r1_v, c1_v, buf0, buf1, acc_sh,
                   sem0, sem1):
        c = lax.axis_index("c")
        t = lax.axis_index("s")

        @pl.loop(0, CHUNK)
        def _(i):
            @pl.loop(0, D_HALF, step=16)
            def _(j):
                buf0[i, pl.ds(j, 16)] = jnp.zeros((16,), jnp.float32)

        @pl.loop(0, SLAB // CHUNK)
        def _(j):
            pltpu.sync_copy(buf0,
                            acc_sh.at[pl.ds(t * SLAB + j * CHUNK, CHUNK)])
        plsc.subcore_barrier()

        def run(y_hbm):
            def stage_idx(p, r_v, c_v):
                @pl.loop(0, CHUNK, step=16)
                def _(i):
                    r_v[pl.ds(i, 16)] = ib[p, 0, pl.ds(i, 16)]
                    c_v[pl.ds(i, 16)] = ib[p, 1, pl.ds(i, 16)]

            def gather(r_v, buf, sem):
                pltpu.async_copy(y_hbm.at[r_v], buf, sem)

            def drain(buf, sem):
                pltpu.make_async_copy(y_hbm.at[pl.ds(0, CHUNK)], buf,
                                      sem).wait()

            def scatter(c_v, buf):
                pltpu.sync_copy(buf, acc_sh.at[c_v], add=True)

            @pl.loop(0, n_blk)
            def _(b):
                pltpu.sync_copy(
                    rc_hbm.at[pl.ds(t * n_chunks + b * NB, NB)], ib)
                stage_idx(0, r0_v, c0_v)
                gather(r0_v, buf0, sem0)
                stage_idx(1, r1_v, c1_v)
                gather(r1_v, buf1, sem1)

                @pl.loop(0, NB // 2)
                def _(j):
                    p0 = 2 * j
                    drain(buf0, sem0)
                    scatter(c0_v, buf0)

                    @pl.when(j < NB // 2 - 1)
                    def _():
                        stage_idx(p0 + 2, r0_v, c0_v)
                        gather(r0_v, buf0, sem0)

                    drain(buf1, sem1)
                    scatter(c1_v, buf1)

                    @pl.when(j < NB // 2 - 1)
                    def _():
                        stage_idx(p0 + 3, r1_v, c1_v)
                        gather(r1_v, buf1, sem1)

        @pl.when(c == 0)
        def _():
            run(y0_hbm)

        @pl.when(c == 1)
        def _():
            run(y1_hbm)

        plsc.subcore_barrier()

        def writeback(a_hbm):
            pltpu.sync_copy(acc_sh.at[pl.ds(t * SLAB, SLAB)],
                            a_hbm.at[pl.ds(t * SLAB, SLAB)])

        @pl.when(c == 0)
        def _():
            writeback(a0_hbm)

        @pl.when(c == 1)
        def _():
            writeback(a1_hbm)

    return agg_kernel(y0, y1, rc3)


_BLK = 1000
_GRID = N_NODES // _BLK

_DOT = functools.partial(jnp.dot, preferred_element_type=jnp.float32,
                         precision=lax.Precision.HIGHEST)


def _row_spec(width):
    return pl.BlockSpec((_BLK, width), lambda i: (i, 0))


def _full_spec(shape):
    return pl.BlockSpec(shape, lambda i: (0, 0))


def _tc_matmul(x, w):
    k_dim, m_dim = w.shape

    def body(x_ref, w_ref, o_ref):
        o_ref[...] = _DOT(x_ref[...], w_ref[...])

    return pl.pallas_call(
        body,
        grid=(_GRID,),
        in_specs=[_row_spec(k_dim), _full_spec((k_dim, m_dim))],
        out_specs=_row_spec(m_dim),
        out_shape=jax.ShapeDtypeStruct((N_NODES, m_dim), jnp.float32),
    )(x, w)


def _tc_scale_split(z, p0, p1):
    def body(z_ref, p0_ref, p1_ref, y0_ref, y1_ref):
        s = lax.rsqrt(p0_ref[:, :1] + p1_ref[:, :1] + 1.0)
        y = s * z_ref[...]
        y0_ref[...] = y[:, :D_HALF]
        y1_ref[...] = y[:, D_HALF:]

    return pl.pallas_call(
        body,
        grid=(_GRID,),
        in_specs=[_row_spec(D_HID), _row_spec(D_DEG), _row_spec(D_DEG)],
        out_specs=(_row_spec(D_HALF), _row_spec(D_HALF)),
        out_shape=(jax.ShapeDtypeStruct((N_NODES, D_HALF), jnp.float32),
                   jax.ShapeDtypeStruct((N_NODES, D_HALF), jnp.float32)),
    )(z, p0, p1)


def _tc_mid(a0, a1, y0, y1, p0, p1, b1, w2):
    def body(a0_ref, a1_ref, y0_ref, y1_ref, p0_ref, p1_ref, b_ref, w_ref,
             o0_ref, o1_ref):
        s = lax.rsqrt(p0_ref[:, :1] + p1_ref[:, :1] + 1.0)
        agg = jnp.concatenate([a0_ref[...], a1_ref[...]], axis=1)
        y = jnp.concatenate([y0_ref[...], y1_ref[...]], axis=1)
        h = jnp.maximum(s * (agg + y) + b_ref[...], 0.0)
        y2 = s * _DOT(h, w_ref[...])
        o0_ref[...] = y2[:, :D_HALF]
        o1_ref[...] = y2[:, D_HALF:]

    return pl.pallas_call(
        body,
        grid=(_GRID,),
        in_specs=[_row_spec(D_HALF), _row_spec(D_HALF),
                  _row_spec(D_HALF), _row_spec(D_HALF),
                  _row_spec(D_DEG), _row_spec(D_DEG),
                  _full_spec((1, D_HID)), _full_spec((D_HID, D_HID))],
        out_specs=(_row_spec(D_HALF), _row_spec(D_HALF)),
        out_shape=(jax.ShapeDtypeStruct((N_NODES, D_HALF), jnp.float32),
                   jax.ShapeDtypeStruct((N_NODES, D_HALF), jnp.float32)),
    )(a0, a1, y0, y1, p0, p1, b1, w2)


def _tc_final(a0, a1, y0, y1, p0, p1, b2, wfc_pad, bfc_pad):
    def body(a0_ref, a1_ref, y0_ref, y1_ref, p0_ref, p1_ref, b_ref, w_ref,
             bf_ref, o_ref):
        s = lax.rsqrt(p0_ref[:, :1] + p1_ref[:, :1] + 1.0)
        agg = jnp.concatenate([a0_ref[...], a1_ref[...]], axis=1)
        y = jnp.concatenate([y0_ref[...], y1_ref[...]], axis=1)
        h = jnp.maximum(s * (agg + y) + b_ref[...], 0.0)
        o_ref[...] = jax.nn.sigmoid(_DOT(h, w_ref[...]) + bf_ref[...])

    return pl.pallas_call(
        body,
        grid=(_GRID,),
        in_specs=[_row_spec(D_HALF), _row_spec(D_HALF),
                  _row_spec(D_HALF), _row_spec(D_HALF),
                  _row_spec(D_DEG), _row_spec(D_DEG),
                  _full_spec((1, D_HID)), _full_spec((D_HID, 128)),
                  _full_spec((1, 128))],
        out_specs=_row_spec(128),
        out_shape=jax.ShapeDtypeStruct((N_NODES, 128), jnp.float32),
    )(a0, a1, y0, y1, p0, p1, b2, wfc_pad, bfc_pad)


def kernel(x, edge_index, W1, b1, W2, b2, Wfc, bfc):
    n_edges = edge_index.shape[1]
    unit = NB * CHUNK * NUM_TILES
    e_pad = -(-n_edges // unit) * unit
    pad_n = e_pad - n_edges

    row = edge_index[0].astype(jnp.int32)
    col = edge_index[1].astype(jnp.int32)
    pad_ids = jnp.arange(pad_n, dtype=jnp.int32) % 16
    row_p = jnp.concatenate([row, pad_ids])
    col_p = jnp.concatenate([col, N_NODES + pad_ids])
    rc3 = jnp.stack([row_p.reshape(-1, CHUNK), col_p.reshape(-1, CHUNK)],
                    axis=1)

    p = _sc_degree(col_p, e_pad)
    p0 = p[:ACC_ROWS]
    p1 = p[ACC_ROWS:]

    z1 = _tc_matmul(x, W1)
    y1_0, y1_1 = _tc_scale_split(z1, p0, p1)
    a1_0, a1_1 = _sc_aggregate(y1_0, y1_1, rc3, e_pad)

    y2_0, y2_1 = _tc_mid(a1_0, a1_1, y1_0, y1_1, p0, p1,
                         b1.reshape(1, D_HID), W2)
    a2_0, a2_1 = _sc_aggregate(y2_0, y2_1, rc3, e_pad)

    wfc_pad = jnp.pad(Wfc, ((0, 0), (0, 128 - Wfc.shape[1])))
    bfc_pad = jnp.pad(bfc.reshape(1, 1), ((0, 0), (0, 127)))
    out = _tc_final(a2_0, a2_1, y2_0, y2_1, p0, p1,
                    b2.reshape(1, D_HID), wfc_pad, bfc_pad)
    return out[:, :1]


# speedup vs baseline: 21.0686x; 1.0183x over previous
import functools

import jax
import jax.numpy as jnp
from jax import lax
from jax.experimental import pallas as pl
from jax.experimental.pallas import tpu as pltpu
from jax.experimental.pallas import tpu_sc as plsc

N_NODES = 10000
D_IN = 128
D_HID = 256
D_HALF = 128

NUM_CORES = 2
NUM_TILES = 16
CHUNK = 128

ACC_ROWS = 10240
SLAB = ACC_ROWS // NUM_TILES


def _sc_mesh():
    return plsc.VectorSubcoreMesh(
        core_axis_name="c", subcore_axis_name="s",
        num_cores=NUM_CORES, num_subcores=NUM_TILES)


D_DEG = 16


def _sc_degree(col_p, e_pad):
    per_worker = e_pad // (NUM_CORES * NUM_TILES)
    n_chunks = per_worker // CHUNK

    @functools.partial(
        pl.kernel,
        out_type=jax.ShapeDtypeStruct((NUM_CORES * ACC_ROWS, D_DEG),
                                      jnp.float32),
        mesh=_sc_mesh(),
        scratch_types=[
            pltpu.VMEM((per_worker,), jnp.int32),
            pltpu.VMEM((CHUNK,), jnp.int32),
            pltpu.VMEM((CHUNK, D_DEG), jnp.float32),
            pltpu.VMEM_SHARED((ACC_ROWS, D_DEG), jnp.float32),
        ],
    )
    def deg_kernel(col_hbm, out_hbm, cidx_v, cchunk_v, ones_v, acc_sh):
        c = lax.axis_index("c")
        t = lax.axis_index("s")
        wid = c * NUM_TILES + t

        @pl.loop(0, CHUNK)
        def _(i):
            ones_v[i, pl.ds(0, D_DEG)] = jnp.zeros((D_DEG,), jnp.float32)

        @pl.loop(0, SLAB // CHUNK)
        def _(j):
            pltpu.sync_copy(ones_v,
                            acc_sh.at[pl.ds(t * SLAB + j * CHUNK, CHUNK)])

        @pl.loop(0, CHUNK)
        def _(i):
            ones_v[i, pl.ds(0, D_DEG)] = jnp.ones((D_DEG,), jnp.float32)

        pltpu.sync_copy(col_hbm.at[pl.ds(wid * per_worker, per_worker)],
                        cidx_v)
        plsc.subcore_barrier()

        @pl.loop(0, n_chunks)
        def _(k):
            @pl.loop(0, CHUNK, step=16)
            def _(i):
                cchunk_v[pl.ds(i, 16)] = cidx_v[pl.ds(k * CHUNK + i, 16)]

            pltpu.sync_copy(ones_v, acc_sh.at[cchunk_v], add=True)

        plsc.subcore_barrier()
        pltpu.sync_copy(
            acc_sh.at[pl.ds(t * SLAB, SLAB)],
            out_hbm.at[pl.ds(c * ACC_ROWS + t * SLAB, SLAB)])

    return deg_kernel(col_p)


NB = 16


def _sc_aggregate(y0, y1, rc3, e_pad):
    per_tile = e_pad // NUM_TILES
    n_chunks = per_tile // CHUNK
    n_blk = n_chunks // NB

    @functools.partial(
        pl.kernel,
        out_type=(jax.ShapeDtypeStruct((ACC_ROWS, D_HALF), jnp.float32),
                  jax.ShapeDtypeStruct((ACC_ROWS, D_HALF), jnp.float32)),
        mesh=_sc_mesh(),
        scratch_types=[
            pltpu.VMEM((NB, 2, CHUNK), jnp.int32),
            pltpu.VMEM((NB, 2, CHUNK), jnp.int32),
            pltpu.VMEM((CHUNK,), jnp.int32),
            pltpu.VMEM((CHUNK,), jnp.int32),
            pltpu.VMEM((CHUNK,), jnp.int32),
            pltpu.VMEM((CHUNK,), jnp.int32),
            pltpu.VMEM((CHUNK, D_HALF), jnp.float32),
            pltpu.VMEM((CHUNK, D_HALF), jnp.float32),
            pltpu.VMEM_SHARED((ACC_ROWS, D_HALF), jnp.float32),
            pltpu.SemaphoreType.DMA,
            pltpu.SemaphoreType.DMA,
            pltpu.SemaphoreType.DMA,
            pltpu.SemaphoreType.DMA,
        ],
    )
    def agg_kernel(y0_hbm, y1_hbm, rc_hbm, a0_hbm, a1_hbm,
                   iba, ibb, r0_v, c0_v, r1_v, c1_v, buf0, buf1, acc_sh,
                   sem0, sem1, semia, semib):
        c = lax.axis_index("c")
        t = lax.axis_index("s")

        def prefetch_ib(b, ibuf, semi):
            pltpu.async_copy(
                rc_hbm.at[pl.ds(t * n_chunks + b * NB, NB)], ibuf, semi)

        def drain_ib(ibuf, semi):
            pltpu.make_async_copy(rc_hbm.at[pl.ds(0, NB)], ibuf,
                                  semi).wait()

        prefetch_ib(0, iba, semia)

        @pl.loop(0, CHUNK)
        def _(i):
            @pl.loop(0, D_HALF, step=16)
            def _(j):
                buf0[i, pl.ds(j, 16)] = jnp.zeros((16,), jnp.float32)

        @pl.loop(0, SLAB // CHUNK)
        def _(j):
            pltpu.sync_copy(buf0,
                            acc_sh.at[pl.ds(t * SLAB + j * CHUNK, CHUNK)])
        plsc.subcore_barrier()

        def run(y_hbm):
            def stage_idx(ib, p, r_v, c_v):
                @pl.loop(0, CHUNK, step=16)
                def _(i):
                    r_v[pl.ds(i, 16)] = ib[p, 0, pl.ds(i, 16)]
                    c_v[pl.ds(i, 16)] = ib[p, 1, pl.ds(i, 16)]

            def gather(r_v, buf, sem):
                pltpu.async_copy(y_hbm.at[r_v], buf, sem)

            def drain(buf, sem):
                pltpu.make_async_copy(y_hbm.at[pl.ds(0, CHUNK)], buf,
                                      sem).wait()

            def scatter(c_v, buf):
                pltpu.sync_copy(buf, acc_sh.at[c_v], add=True)

            def process(ib):
                stage_idx(ib, 0, r0_v, c0_v)
                gather(r0_v, buf0, sem0)
                stage_idx(ib, 1, r1_v, c1_v)
                gather(r1_v, buf1, sem1)

                @pl.loop(0, NB // 2)
                def _(j):
                    p0 = 2 * j
                    drain(buf0, sem0)
                    scatter(c0_v, buf0)

                    @pl.when(j < NB // 2 - 1)
                    def _():
                        stage_idx(ib, p0 + 2, r0_v, c0_v)
                        gather(r0_v, buf0, sem0)

                    drain(buf1, sem1)
                    scatter(c1_v, buf1)

                    @pl.when(j < NB // 2 - 1)
                    def _():
                        stage_idx(ib, p0 + 3, r1_v, c1_v)
                        gather(r1_v, buf1, sem1)

            @pl.loop(0, n_blk, step=2)
            def _(b):
                prefetch_ib(b + 1, ibb, semib)
                drain_ib(iba, semia)
                process(iba)

                @pl.when(b + 2 < n_blk)
                def _():
                    prefetch_ib(b + 2, iba, semia)

                drain_ib(ibb, semib)
                process(ibb)

        @pl.when(c == 0)
        def _():
            run(y0_hbm)

        @pl.when(c == 1)
        def _():
            run(y1_hbm)

        plsc.subcore_barrier()

        def writeback(a_hbm):
            pltpu.sync_copy(acc_sh.at[pl.ds(t * SLAB, SLAB)],
                            a_hbm.at[pl.ds(t * SLAB, SLAB)])

        @pl.when(c == 0)
        def _():
            writeback(a0_hbm)

        @pl.when(c == 1)
        def _():
            writeback(a1_hbm)

    return agg_kernel(y0, y1, rc3)


_BLK = 1000
_GRID = N_NODES // _BLK

_DOT = functools.partial(jnp.dot, preferred_element_type=jnp.float32,
                         precision=lax.Precision.HIGHEST)


def _row_spec(width):
    return pl.BlockSpec((_BLK, width), lambda i: (i, 0))


def _full_spec(shape):
    return pl.BlockSpec(shape, lambda i: (0, 0))


def _tc_matmul(x, w):
    k_dim, m_dim = w.shape

    def body(x_ref, w_ref, o_ref):
        o_ref[...] = _DOT(x_ref[...], w_ref[...])

    return pl.pallas_call(
        body,
        grid=(_GRID,),
        in_specs=[_row_spec(k_dim), _full_spec((k_dim, m_dim))],
        out_specs=_row_spec(m_dim),
        out_shape=jax.ShapeDtypeStruct((N_NODES, m_dim), jnp.float32),
    )(x, w)


def _tc_scale_split(z, p0, p1):
    def body(z_ref, p0_ref, p1_ref, y0_ref, y1_ref):
        s = lax.rsqrt(p0_ref[:, :1] + p1_ref[:, :1] + 1.0)
        y = s * z_ref[...]
        y0_ref[...] = y[:, :D_HALF]
        y1_ref[...] = y[:, D_HALF:]

    return pl.pallas_call(
        body,
        grid=(_GRID,),
        in_specs=[_row_spec(D_HID), _row_spec(D_DEG), _row_spec(D_DEG)],
        out_specs=(_row_spec(D_HALF), _row_spec(D_HALF)),
        out_shape=(jax.ShapeDtypeStruct((N_NODES, D_HALF), jnp.float32),
                   jax.ShapeDtypeStruct((N_NODES, D_HALF), jnp.float32)),
    )(z, p0, p1)


def _tc_mid(a0, a1, y0, y1, p0, p1, b1, w2):
    def body(a0_ref, a1_ref, y0_ref, y1_ref, p0_ref, p1_ref, b_ref, w_ref,
             o0_ref, o1_ref):
        s = lax.rsqrt(p0_ref[:, :1] + p1_ref[:, :1] + 1.0)
        agg = jnp.concatenate([a0_ref[...], a1_ref[...]], axis=1)
        y = jnp.concatenate([y0_ref[...], y1_ref[...]], axis=1)
        h = jnp.maximum(s * (agg + y) + b_ref[...], 0.0)
        y2 = s * _DOT(h, w_ref[...])
        o0_ref[...] = y2[:, :D_HALF]
        o1_ref[...] = y2[:, D_HALF:]

    return pl.pallas_call(
        body,
        grid=(_GRID,),
        in_specs=[_row_spec(D_HALF), _row_spec(D_HALF),
                  _row_spec(D_HALF), _row_spec(D_HALF),
                  _row_spec(D_DEG), _row_spec(D_DEG),
                  _full_spec((1, D_HID)), _full_spec((D_HID, D_HID))],
        out_specs=(_row_spec(D_HALF), _row_spec(D_HALF)),
        out_shape=(jax.ShapeDtypeStruct((N_NODES, D_HALF), jnp.float32),
                   jax.ShapeDtypeStruct((N_NODES, D_HALF), jnp.float32)),
    )(a0, a1, y0, y1, p0, p1, b1, w2)


def _tc_final(a0, a1, y0, y1, p0, p1, b2, wfc_pad, bfc_pad):
    def body(a0_ref, a1_ref, y0_ref, y1_ref, p0_ref, p1_ref, b_ref, w_ref,
             bf_ref, o_ref):
        s = lax.rsqrt(p0_ref[:, :1] + p1_ref[:, :1] + 1.0)
        agg = jnp.concatenate([a0_ref[...], a1_ref[...]], axis=1)
        y = jnp.concatenate([y0_ref[...], y1_ref[...]], axis=1)
        h = jnp.maximum(s * (agg + y) + b_ref[...], 0.0)
        o_ref[...] = jax.nn.sigmoid(_DOT(h, w_ref[...]) + bf_ref[...])

    return pl.pallas_call(
        body,
        grid=(_GRID,),
        in_specs=[_row_spec(D_HALF), _row_spec(D_HALF),
                  _row_spec(D_HALF), _row_spec(D_HALF),
                  _row_spec(D_DEG), _row_spec(D_DEG),
                  _full_spec((1, D_HID)), _full_spec((D_HID, 128)),
                  _full_spec((1, 128))],
        out_specs=_row_spec(128),
        out_shape=jax.ShapeDtypeStruct((N_NODES, 128), jnp.float32),
    )(a0, a1, y0, y1, p0, p1, b2, wfc_pad, bfc_pad)


def kernel(x, edge_index, W1, b1, W2, b2, Wfc, bfc):
    n_edges = edge_index.shape[1]
    unit = 2 * NB * CHUNK * NUM_TILES
    e_pad = -(-n_edges // unit) * unit
    pad_n = e_pad - n_edges

    row = edge_index[0].astype(jnp.int32)
    col = edge_index[1].astype(jnp.int32)
    pad_ids = jnp.arange(pad_n, dtype=jnp.int32) % 16
    row_p = jnp.concatenate([row, pad_ids])
    col_p = jnp.concatenate([col, N_NODES + pad_ids])
    rc3 = jnp.stack([row_p.reshape(-1, CHUNK), col_p.reshape(-1, CHUNK)],
                    axis=1)

    p = _sc_degree(col_p, e_pad)
    p0 = p[:ACC_ROWS]
    p1 = p[ACC_ROWS:]

    z1 = _tc_matmul(x, W1)
    y1_0, y1_1 = _tc_scale_split(z1, p0, p1)
    a1_0, a1_1 = _sc_aggregate(y1_0, y1_1, rc3, e_pad)

    y2_0, y2_1 = _tc_mid(a1_0, a1_1, y1_0, y1_1, p0, p1,
                         b1.reshape(1, D_HID), W2)
    a2_0, a2_1 = _sc_aggregate(y2_0, y2_1, rc3, e_pad)

    wfc_pad = jnp.pad(Wfc, ((0, 0), (0, 128 - Wfc.shape[1])))
    bfc_pad = jnp.pad(bfc.reshape(1, 1), ((0, 0), (0, 127)))
    out = _tc_final(a2_0, a2_1, y2_0, y2_1, p0, p1,
                    b2.reshape(1, D_HID), wfc_pad, bfc_pad)
    return out[:, :1]
